# baseline (speedup 1.0000x reference)
import functools

import jax
import jax.numpy as jnp
import numpy as np
from jax import lax
from jax.experimental import pallas as pl
from jax.experimental.pallas import tpu as pltpu
from jax.experimental.pallas import tpu_sc as plsc

_CLASS_TOKENS = (11, 257, 1024, 4096, 9999, 20000, 50000, 99999)
_NUM_CLASSES = len(_CLASS_TOKENS)
_L = 16
_LN2 = float(np.log(2.0))

_GATHER_DNUMS = lax.GatherDimensionNumbers(
    offset_dims=(), collapsed_slice_dims=(0,), start_index_map=(0,))


def _shuffle(v, perm):
    return lax.gather(v, perm[:, None], _GATHER_DNUMS, slice_sizes=(1,),
                      mode=lax.GatherScatterMode.PROMISE_IN_BOUNDS)


@functools.lru_cache(maxsize=None)
def _build_sc_call(B, S, V):
    assert B == 32 and V > max(_CLASS_TOKENS)
    bases = [t & ~(_L - 1) for t in _CLASS_TOKENS]
    lanes = [t & (_L - 1) for t in _CLASS_TOKENS]

    mesh = plsc.VectorSubcoreMesh(core_axis_name="c", subcore_axis_name="s")
    info = plsc.get_sparse_core_info()
    nc = info.num_cores

    @functools.partial(
        pl.kernel,
        mesh=mesh,
        out_type=jax.ShapeDtypeStruct((B,), jnp.float32),
        scratch_types=(
            [pltpu.VMEM((_L,), jnp.float32) for _ in range(_NUM_CLASSES)]
            + [
                pltpu.VMEM((B,), jnp.int32),
                pltpu.VMEM((_L,), jnp.float32),
                pltpu.SemaphoreType.DMA,
                pltpu.SemaphoreType.DMA,
            ]
        ),
    )
    def sc_call(mo_hbm, tgt_hbm, out_hbm, *rest):
        bufs = rest[:_NUM_CLASSES]
        tgt_v, out_v, sem, sem2 = rest[_NUM_CLASSES:]
        wid = lax.axis_index("s") * nc + lax.axis_index("c")
        widv = jnp.full((_L,), wid, dtype=jnp.int32)
        copies = [
            pltpu.async_copy(mo_hbm.at[wid, S - 1, pl.ds(bases[j], _L)],
                             bufs[j], sem)
            for j in range(_NUM_CLASSES)
        ]
        tgt_copy = pltpu.async_copy(tgt_hbm, tgt_v, sem2)
        for c in copies:
            c.wait()

        picks = [
            _shuffle(bufs[j][...], jnp.full((_L,), lanes[j], jnp.int32))
            for j in range(_NUM_CLASSES)
        ]
        m = picks[0]
        for p in picks[1:]:
            m = jnp.maximum(m, p)
        s = jnp.exp(picks[0] - m)
        for p in picks[1:]:
            s = s + jnp.exp(p - m)
        bits = lax.bitcast_convert_type(s, jnp.int32)
        e = ((bits >> 23) - 127).astype(jnp.float32)
        f = lax.bitcast_convert_type((bits & 0x007FFFFF) | 0x3F800000,
                                     jnp.float32)
        r = (f - 1.0) / (f + 1.0)
        r2 = r * r
        log_f = 2.0 * r * (1.0 + r2 * (1.0 / 3.0 + r2 * (
            1.0 / 5.0 + r2 * (1.0 / 9.0 * r2 + 1.0 / 7.0))))
        lse_v = m + e * _LN2 + log_f

        tgt_copy.wait()
        widm = widv & (_L - 1)
        tgt_splat = jnp.where(widv < _L,
                              _shuffle(tgt_v[pl.ds(0, _L)], widm),
                              _shuffle(tgt_v[pl.ds(_L, _L)], widm))
        picked = picks[0]
        for j in range(1, _NUM_CLASSES):
            picked = jnp.where(tgt_splat == j, picks[j], picked)

        out_v[...] = lse_v - picked
        pltpu.async_copy(out_v, out_hbm.at[widv], sem2).wait()

    def run(model_outputs, targets):
        return sc_call(model_outputs, targets.reshape(B).astype(jnp.int32))

    return run


def kernel(model_outputs, targets, input_pos):
    B, S, V = model_outputs.shape
    return _build_sc_call(B, S, V)(model_outputs, targets)


# speedup vs baseline: 4.5298x; 4.5298x over previous
import functools

import jax
import jax.numpy as jnp
import numpy as np
from jax import lax
from jax.experimental import pallas as pl
from jax.experimental.pallas import tpu as pltpu
from jax.experimental.pallas import tpu_sc as plsc

_CLASS_TOKENS = (11, 257, 1024, 4096, 9999, 20000, 50000, 99999)
_NUM_CLASSES = len(_CLASS_TOKENS)
_L = 16
_LN2 = float(np.log(2.0))

_GATHER_DNUMS = lax.GatherDimensionNumbers(
    offset_dims=(), collapsed_slice_dims=(0,), start_index_map=(0,))


def _shuffle(v, perm):
    return lax.gather(v, perm[:, None], _GATHER_DNUMS, slice_sizes=(1,),
                      mode=lax.GatherScatterMode.PROMISE_IN_BOUNDS)


@functools.lru_cache(maxsize=None)
def _build_sc_call(B, S, V):
    assert B == 32 and V > max(_CLASS_TOKENS)
    bases = [t & ~(_L - 1) for t in _CLASS_TOKENS]
    lanes = [t & (_L - 1) for t in _CLASS_TOKENS]

    mesh = plsc.VectorSubcoreMesh(core_axis_name="c", subcore_axis_name="s")
    info = plsc.get_sparse_core_info()
    nc = info.num_cores

    @functools.partial(
        pl.kernel,
        mesh=mesh,
        out_type=jax.ShapeDtypeStruct((B, _L), jnp.float32),
        scratch_types=(
            [pltpu.VMEM((_L,), jnp.float32) for _ in range(_NUM_CLASSES)]
            + [
                pltpu.VMEM((B,), jnp.int32),
                pltpu.VMEM((_L,), jnp.float32),
                pltpu.SemaphoreType.DMA,
                pltpu.SemaphoreType.DMA,
            ]
        ),
    )
    def sc_call(mo_hbm, tgt_hbm, out_hbm, *rest):
        bufs = rest[:_NUM_CLASSES]
        tgt_v, out_v, sem, sem2 = rest[_NUM_CLASSES:]
        wid = lax.axis_index("s") * nc + lax.axis_index("c")
        widv = jnp.full((_L,), wid, dtype=jnp.int32)
        copies = [
            pltpu.async_copy(mo_hbm.at[wid, S - 1, pl.ds(bases[j], _L)],
                             bufs[j], sem)
            for j in range(_NUM_CLASSES)
        ]
        tgt_copy = pltpu.async_copy(tgt_hbm, tgt_v, sem2)
        for c in copies:
            c.wait()

        picks = [
            _shuffle(bufs[j][...], jnp.full((_L,), lanes[j], jnp.int32))
            for j in range(_NUM_CLASSES)
        ]
        m = picks[0]
        for p in picks[1:]:
            m = jnp.maximum(m, p)
        s = jnp.exp(picks[0] - m)
        for p in picks[1:]:
            s = s + jnp.exp(p - m)
        bits = lax.bitcast_convert_type(s, jnp.int32)
        e = ((bits >> 23) - 127).astype(jnp.float32)
        f = lax.bitcast_convert_type((bits & 0x007FFFFF) | 0x3F800000,
                                     jnp.float32)
        r = (f - 1.0) / (f + 1.0)
        r2 = r * r
        log_f = 2.0 * r * (1.0 + r2 * (1.0 / 3.0 + r2 * (
            1.0 / 5.0 + r2 * (1.0 / 9.0 * r2 + 1.0 / 7.0))))
        lse_v = m + e * _LN2 + log_f

        tgt_copy.wait()
        widm = widv & (_L - 1)
        tgt_splat = jnp.where(widv < _L,
                              _shuffle(tgt_v[pl.ds(0, _L)], widm),
                              _shuffle(tgt_v[pl.ds(_L, _L)], widm))
        picked = picks[0]
        for j in range(1, _NUM_CLASSES):
            picked = jnp.where(tgt_splat == j, picks[j], picked)

        out_v[...] = lse_v - picked
        pltpu.sync_copy(out_v, out_hbm.at[wid])

    def run(model_outputs, targets):
        out2d = sc_call(model_outputs, targets.reshape(B).astype(jnp.int32))
        return out2d[:, 0]

    return run


def kernel(model_outputs, targets, input_pos):
    B, S, V = model_outputs.shape
    return _build_sc_call(B, S, V)(model_outputs, targets)


# speedup vs baseline: 4.7052x; 1.0387x over previous
import functools

import jax
import jax.numpy as jnp
import numpy as np
from jax import lax
from jax.experimental import pallas as pl
from jax.experimental.pallas import tpu as pltpu
from jax.experimental.pallas import tpu_sc as plsc

_CLASS_TOKENS = (11, 257, 1024, 4096, 9999, 20000, 50000, 99999)
_NUM_CLASSES = len(_CLASS_TOKENS)
_L = 16
_LN2 = float(np.log(2.0))
_NCORES = 1

_GATHER_DNUMS = lax.GatherDimensionNumbers(
    offset_dims=(), collapsed_slice_dims=(0,), start_index_map=(0,))


def _shuffle(v, perm):
    return lax.gather(v, perm[:, None], _GATHER_DNUMS, slice_sizes=(1,),
                      mode=lax.GatherScatterMode.PROMISE_IN_BOUNDS)


def _cross_entropy(picks, tgt_splat):
    m = picks[0]
    for p in picks[1:]:
        m = jnp.maximum(m, p)
    s = jnp.exp(picks[0] - m)
    for p in picks[1:]:
        s = s + jnp.exp(p - m)
    bits = lax.bitcast_convert_type(s, jnp.int32)
    e = ((bits >> 23) - 127).astype(jnp.float32)
    f = lax.bitcast_convert_type((bits & 0x007FFFFF) | 0x3F800000,
                                 jnp.float32)
    r = (f - 1.0) / (f + 1.0)
    r2 = r * r
    log_f = 2.0 * r * (1.0 + r2 * (1.0 / 3.0 + r2 * (
        1.0 / 5.0 + r2 * (1.0 / 9.0 * r2 + 1.0 / 7.0))))
    lse_v = m + e * _LN2 + log_f

    picked = picks[0]
    for j in range(1, _NUM_CLASSES):
        picked = jnp.where(tgt_splat == j, picks[j], picked)
    return lse_v - picked


@functools.lru_cache(maxsize=None)
def _build_sc_call(B, S, V):
    assert V > max(_CLASS_TOKENS)
    bases = [t & ~(_L - 1) for t in _CLASS_TOKENS]
    lanes = [t & (_L - 1) for t in _CLASS_TOKENS]

    info = plsc.get_sparse_core_info()
    nw = _NCORES * info.num_subcores
    assert B % nw == 0
    bpw = B // nw
    mesh = plsc.VectorSubcoreMesh(core_axis_name="c", subcore_axis_name="s",
                                  num_cores=_NCORES)

    @functools.partial(
        pl.kernel,
        mesh=mesh,
        out_type=jax.ShapeDtypeStruct((B, _L), jnp.float32),
        scratch_types=(
            [pltpu.VMEM((_L,), jnp.float32)
             for _ in range(_NUM_CLASSES * bpw)]
            + [
                pltpu.VMEM((B,), jnp.int32),
                pltpu.VMEM((_L,), jnp.float32),
                pltpu.SemaphoreType.DMA,
                pltpu.SemaphoreType.DMA,
            ]
        ),
    )
    def sc_call(mo_hbm, tgt_hbm, out_hbm, *rest):
        bufs = rest[:_NUM_CLASSES * bpw]
        tgt_v, out_v, sem, sem2 = rest[_NUM_CLASSES * bpw:]
        wid = lax.axis_index("s") * _NCORES + lax.axis_index("c")
        lane = lax.iota(jnp.int32, _L)

        copies = []
        for k in range(bpw):
            b = wid + k * nw
            for j in range(_NUM_CLASSES):
                copies.append(pltpu.async_copy(
                    mo_hbm.at[b, S - 1, pl.ds(bases[j], _L)],
                    bufs[k * _NUM_CLASSES + j], sem))
        tgt_copy = pltpu.async_copy(tgt_hbm, tgt_v, sem2)
        for c in copies:
            c.wait()
        tgt_copy.wait()

        for k in range(bpw):
            b = wid + k * nw
            bv = jnp.full((_L,), b, dtype=jnp.int32)
            picks = [
                _shuffle(bufs[k * _NUM_CLASSES + j][...],
                         jnp.full((_L,), lanes[j], jnp.int32))
                for j in range(_NUM_CLASSES)
            ]
            bm = bv & (_L - 1)
            tgt_splat = jnp.where(bv < _L,
                                  _shuffle(tgt_v[pl.ds(0, _L)], bm),
                                  _shuffle(tgt_v[pl.ds(_L, _L)], bm))
            out_v[...] = _cross_entropy(picks, tgt_splat)
            pltpu.sync_copy(out_v, out_hbm.at[b])

    def run(model_outputs, targets):
        out2d = sc_call(model_outputs, targets.reshape(B).astype(jnp.int32))
        return out2d[:, 0]

    return run


def kernel(model_outputs, targets, input_pos):
    B, S, V = model_outputs.shape
    return _build_sc_call(B, S, V)(model_outputs, targets)


# speedup vs baseline: 4.7974x; 1.0196x over previous
import functools

import jax
import jax.numpy as jnp
import numpy as np
from jax import lax
from jax.experimental import pallas as pl
from jax.experimental.pallas import tpu as pltpu
from jax.experimental.pallas import tpu_sc as plsc

_CLASS_TOKENS = (11, 257, 1024, 4096, 9999, 20000, 50000, 99999)
_NUM_CLASSES = len(_CLASS_TOKENS)
_L = 16
_LN2 = float(np.log(2.0))
_NCORES = 1

_GATHER_DNUMS = lax.GatherDimensionNumbers(
    offset_dims=(), collapsed_slice_dims=(0,), start_index_map=(0,))


def _shuffle(v, perm):
    return lax.gather(v, perm[:, None], _GATHER_DNUMS, slice_sizes=(1,),
                      mode=lax.GatherScatterMode.PROMISE_IN_BOUNDS)


def _cross_entropy(picks, tgt_splat):
    m = picks[0]
    for p in picks[1:]:
        m = jnp.maximum(m, p)
    s = jnp.exp(picks[0] - m)
    for p in picks[1:]:
        s = s + jnp.exp(p - m)
    bits = lax.bitcast_convert_type(s, jnp.int32)
    e = ((bits >> 23) - 127).astype(jnp.float32)
    f = lax.bitcast_convert_type((bits & 0x007FFFFF) | 0x3F800000,
                                 jnp.float32)
    r = (f - 1.0) / (f + 1.0)
    r2 = r * r
    log_f = 2.0 * r * (1.0 + r2 * (1.0 / 3.0 + r2 * (
        1.0 / 5.0 + r2 * (1.0 / 9.0 * r2 + 1.0 / 7.0))))
    lse_v = m + e * _LN2 + log_f

    picked = picks[0]
    for j in range(1, _NUM_CLASSES):
        picked = jnp.where(tgt_splat == j, picks[j], picked)
    return lse_v - picked


@functools.lru_cache(maxsize=None)
def _build_sc_call(B, S, V):
    assert V > max(_CLASS_TOKENS)
    bases = [t & ~(_L - 1) for t in _CLASS_TOKENS]
    lanes = [t & (_L - 1) for t in _CLASS_TOKENS]

    info = plsc.get_sparse_core_info()
    nw = _NCORES * info.num_subcores
    assert B % nw == 0
    bpw = B // nw
    mesh = plsc.VectorSubcoreMesh(core_axis_name="c", subcore_axis_name="s",
                                  num_cores=_NCORES)

    @functools.partial(
        pl.kernel,
        mesh=mesh,
        out_type=jax.ShapeDtypeStruct((B, _L), jnp.float32),
        scratch_types=(
            [pltpu.VMEM((_L,), jnp.float32)
             for _ in range(_NUM_CLASSES * bpw)]
            + [pltpu.VMEM((_L,), jnp.float32) for _ in range(2)]
            + [
                pltpu.VMEM((B,), jnp.int32),
                pltpu.SemaphoreType.DMA,
                pltpu.SemaphoreType.DMA,
                pltpu.SemaphoreType.DMA,
            ]
        ),
    )
    def sc_call(mo_hbm, tgt_hbm, out_hbm, *rest):
        bufs = rest[:_NUM_CLASSES * bpw]
        out_a, out_b, tgt_v, sem, sem2, sem3 = rest[_NUM_CLASSES * bpw:]
        outs = [out_a, out_b]
        wid = lax.axis_index("s") * _NCORES + lax.axis_index("c")

        copies = []
        for k in range(bpw):
            b = wid + k * nw
            for j in range(_NUM_CLASSES):
                copies.append(pltpu.async_copy(
                    mo_hbm.at[b, S - 1, pl.ds(bases[j], _L)],
                    bufs[k * _NUM_CLASSES + j], sem))
        tgt_copy = pltpu.async_copy(tgt_hbm, tgt_v, sem2)
        tgt_copy.wait()

        out_copies = []
        for k in range(bpw):
            b = wid + k * nw
            bv = jnp.full((_L,), b, dtype=jnp.int32)
            for j in range(_NUM_CLASSES):
                copies[k * _NUM_CLASSES + j].wait()
            picks = [
                _shuffle(bufs[k * _NUM_CLASSES + j][...],
                         jnp.full((_L,), lanes[j], jnp.int32))
                for j in range(_NUM_CLASSES)
            ]
            bm = bv & (_L - 1)
            tgt_splat = jnp.where(bv < _L,
                                  _shuffle(tgt_v[pl.ds(0, _L)], bm),
                                  _shuffle(tgt_v[pl.ds(_L, _L)], bm))
            ov = outs[k % 2]
            ov[...] = _cross_entropy(picks, tgt_splat)
            out_copies.append(pltpu.async_copy(ov, out_hbm.at[b], sem3))
        for c in out_copies:
            c.wait()

    def run(model_outputs, targets):
        out2d = sc_call(model_outputs, targets.reshape(B).astype(jnp.int32))
        return out2d[:, 0]

    return run


def kernel(model_outputs, targets, input_pos):
    B, S, V = model_outputs.shape
    return _build_sc_call(B, S, V)(model_outputs, targets)
